# baseline (speedup 1.0000x reference)
import dataclasses
import functools
import math

import jax
import jax.numpy as jnp
from jax import lax
from jax.experimental import pallas as pl
from jax.experimental.pallas import tpu as pltpu
from jax.experimental.pallas import tpu_sc as plsc

NC = 2
NS = 16
NW = NC * NS
DPT = 4


def _sc_aggregate(xr, src, dst, zeros4, zeros1):
    nw, nd = xr.shape
    dpt = DPT
    n = nd // dpt
    e = src.shape[0]
    ch = 2000
    nch = e // ch
    gu = 5
    epw = e // NW

    mesh = plsc.VectorSubcoreMesh(
        core_axis_name="c", subcore_axis_name="s", num_cores=NC,
        num_subcores=NS)

    cp = pltpu.CompilerParams()
    if "needs_layout_passes" in pltpu.CompilerParams.__dataclass_fields__:
        cp = dataclasses.replace(cp, needs_layout_passes=False)

    @functools.partial(
        pl.kernel,
        out_type=(
            jax.ShapeDtypeStruct((NW, n * dpt), jnp.float32),
            jax.ShapeDtypeStruct((NW, n), jnp.float32),
        ),
        mesh=mesh,
        scratch_types=[
            pltpu.VMEM((n * dpt,), jnp.float32),
            pltpu.VMEM((n * dpt,), jnp.float32),
            pltpu.VMEM((n,), jnp.float32),
            pltpu.VMEM((ch,), jnp.int32),
            pltpu.VMEM((ch,), jnp.int32),
            pltpu.VMEM((ch,), jnp.int32),
            pltpu.VMEM((ch,), jnp.int32),
            pltpu.SemaphoreType.DMA,
            pltpu.SemaphoreType.DMA,
            pltpu.SemaphoreType.DMA,
        ],
        compiler_params=cp,
    )
    def sc_agg(xr_hbm, src_hbm, dst_hbm, z4_hbm, z1_hbm, out_hbm, cnt_hbm,
               xs_v, acc_v, cnt_v, sring0_v, sring1_v, dring0_v, dring1_v,
               sem_x, sem_i0, sem_i1):
        c = lax.axis_index("c")
        s = lax.axis_index("s")
        g = c * NS + s
        isems = (sem_i0, sem_i1)
        srings = (sring0_v, sring1_v)
        drings = (dring0_v, dring1_v)

        xcp = pltpu.make_async_copy(xr_hbm.at[g], xs_v, sem_x)
        xcp.start()
        pltpu.sync_copy(z4_hbm, acc_v)
        pltpu.sync_copy(z1_hbm, cnt_v)

        ones = jnp.ones((16,), jnp.float32)

        def idx_copy(j, b):
            return (
                pltpu.make_async_copy(src_hbm.at[pl.ds(j * ch, ch)],
                                      srings[b], isems[b]),
                pltpu.make_async_copy(dst_hbm.at[pl.ds(j * ch, ch)],
                                      drings[b], isems[b]),
            )

        def idx_start(j, b):
            for cp_ in idx_copy(j, b):
                cp_.start()

        def idx_wait(j, b):
            for cp_ in idx_copy(j, b):
                cp_.wait()

        idx_start(0, 0)
        idx_start(1, 1)
        xcp.wait()

        def chunk_body(j, b, pref=True):
            idx_wait(j, b)

            @pl.loop(0, ch // (16 * gu))
            def _(i):
                for u in range(gu):
                    off = (i * gu + u) * 16
                    sv = srings[b][pl.ds(off, 16)] * dpt
                    dv = drings[b][pl.ds(off, 16)] * dpt
                    for dd in range(dpt):
                        v = plsc.load_gather(xs_v, [sv + dd])
                        plsc.addupdate_scatter(acc_v, [dv + dd], v)

            if pref:
                idx_start(j + 2, b)

        @pl.loop(0, nch // 2 - 1)
        def _(p):
            for b in range(2):
                chunk_body(2 * p + b, b)

        chunk_body(nch - 2, 0, pref=False)
        chunk_body(nch - 1, 1, pref=False)

        @pl.loop(0, epw // ch)
        def _(q):
            pltpu.sync_copy(dst_hbm.at[pl.ds(g * epw + q * ch, ch)],
                            dring0_v)

            @pl.loop(0, ch // (16 * gu))
            def _(i):
                for u in range(gu):
                    off = (i * gu + u) * 16
                    dv = dring0_v[pl.ds(off, 16)]
                    plsc.addupdate_scatter(cnt_v, [dv], ones)

        pltpu.sync_copy(acc_v, out_hbm.at[g])
        pltpu.sync_copy(cnt_v, cnt_hbm.at[g])

    return sc_agg(xr, src, dst, zeros4, zeros1)


def _tc_combine(agg_ref, cntp_ref, x_ref, wl_ref, bl_ref, wr_ref, g_ref,
                b_ref, o_ref):
    cnt = jnp.sum(cntp_ref[...], axis=1)
    mean = agg_ref[...] / jnp.maximum(cnt, 1.0)[:, None]
    h = (jnp.dot(mean, wl_ref[...], preferred_element_type=jnp.float32)
         + jnp.dot(x_ref[...], wr_ref[...], preferred_element_type=jnp.float32)
         + bl_ref[...])
    mu = jnp.mean(h, axis=1, keepdims=True)
    hc = h - mu
    var = jnp.mean(hc * hc, axis=1, keepdims=True)
    hn = hc * lax.rsqrt(var + 1e-5) * g_ref[...] + b_ref[...]
    o_ref[...] = 0.5 * hn * (1.0 + lax.erf(hn * (1.0 / math.sqrt(2.0))))


def kernel(x, edge_index, W_l, b_l, W_r, ln_gamma, ln_beta):
    n, d = x.shape
    src = edge_index[0]
    dst = edge_index[1]
    xr = jnp.transpose(x.reshape(n, NW, DPT), (1, 0, 2)).reshape(NW, n * DPT)
    zeros4 = jnp.zeros((n * DPT,), jnp.float32)
    zeros1 = jnp.zeros((n,), jnp.float32)
    out, cntp = _sc_aggregate(xr, src, dst, zeros4, zeros1)
    agg = jnp.transpose(out.reshape(NW, n, DPT), (1, 0, 2)).reshape(n, d)

    blk = 1000
    grid = (n // blk,)
    out = pl.pallas_call(
        _tc_combine,
        grid=grid,
        in_specs=[
            pl.BlockSpec((blk, d), lambda i: (i, 0)),
            pl.BlockSpec((blk, NW), lambda i: (i, 0)),
            pl.BlockSpec((blk, d), lambda i: (i, 0)),
            pl.BlockSpec((d, d), lambda i: (0, 0)),
            pl.BlockSpec((1, d), lambda i: (0, 0)),
            pl.BlockSpec((d, d), lambda i: (0, 0)),
            pl.BlockSpec((1, d), lambda i: (0, 0)),
            pl.BlockSpec((1, d), lambda i: (0, 0)),
        ],
        out_specs=pl.BlockSpec((blk, d), lambda i: (i, 0)),
        out_shape=jax.ShapeDtypeStruct((n, d), jnp.float32),
    )(agg, cntp.T, x, W_l, b_l.reshape(1, d), W_r, ln_gamma.reshape(1, d),
      ln_beta.reshape(1, d))
    return out


# speedup vs baseline: 3.5179x; 3.5179x over previous
import dataclasses
import functools
import math

import jax
import jax.numpy as jnp
from jax import lax
from jax.experimental import pallas as pl
from jax.experimental.pallas import tpu as pltpu
from jax.experimental.pallas import tpu_sc as plsc

NC = 2
NS = 16
NW = NC * NS


def _sc_aggregate(src, dst, x, zeros2d, zeros1d):
    n, d = x.shape
    e = src.shape[0]
    epw = e // NW
    ch = 48
    nfull = epw // ch
    tail_e = epw - nfull * ch
    R = 6
    rps = (n // NS) // 8 * 8
    tail = n - NS * rps

    mesh = plsc.VectorSubcoreMesh(
        core_axis_name="c", subcore_axis_name="s", num_cores=NC,
        num_subcores=NS)

    cp = pltpu.CompilerParams()
    if "needs_layout_passes" in pltpu.CompilerParams.__dataclass_fields__:
        cp = dataclasses.replace(cp, needs_layout_passes=False)

    @functools.partial(
        pl.kernel,
        out_type=(
            jax.ShapeDtypeStruct((NC, n, d), jnp.float32),
            jax.ShapeDtypeStruct((NW, n), jnp.float32),
        ),
        mesh=mesh,
        scratch_types=[
            pltpu.VMEM((R, ch), jnp.int32),
            pltpu.VMEM((R, ch), jnp.int32),
            pltpu.VMEM((1, 16), jnp.int32),
            pltpu.VMEM((R, ch, d), jnp.float32),
            pltpu.VMEM((n,), jnp.float32),
            pltpu.VMEM_SHARED((n, d), jnp.float32),
        ] + [pltpu.SemaphoreType.DMA] * 12,
        compiler_params=cp,
    )
    def sc_agg(src_hbm, dst_hbm, x_hbm, z2_hbm, z1_hbm, part_hbm, cnt_hbm,
               sring_v, dring_v, dtail_v, rows_v, cnt_v, acc_sh, *sems):
        c = lax.axis_index("c")
        s = lax.axis_index("s")
        wid = c * NS + s
        base = wid * epw
        pltpu.sync_copy(z1_hbm, cnt_v)
        pltpu.sync_copy(z2_hbm.at[pl.ds(s * rps, rps)],
                        acc_sh.at[pl.ds(s * rps, rps)])

        @pl.when(s == 0)
        def _():
            pltpu.sync_copy(z2_hbm.at[pl.ds(NS * rps, tail)],
                            acc_sh.at[pl.ds(NS * rps, tail)])

        plsc.subcore_barrier()

        ones = jnp.ones((16,), jnp.float32)
        gsems = sems[:R]
        isems = sems[R:]

        def hist(b, width, ring):
            for k in range(width // 16):
                dk = ring[b, pl.ds(k * 16, 16)]
                plsc.addupdate_scatter(cnt_v, [dk], ones)

        def idx_copy(j, b):
            return (
                pltpu.make_async_copy(src_hbm.at[pl.ds(base + j * ch, ch)],
                                      sring_v.at[b], isems[b]),
                pltpu.make_async_copy(dst_hbm.at[pl.ds(base + j * ch, ch)],
                                      dring_v.at[b], isems[b]),
            )

        def idx_start(j, b):
            for cp_ in idx_copy(j, b):
                cp_.start()

        def idx_wait(j, b):
            for cp_ in idx_copy(j, b):
                cp_.wait()

        def gather_copy(b):
            return pltpu.make_async_copy(
                x_hbm.at[sring_v.at[b]], rows_v.at[b], gsems[b])

        AH = R - 1

        def body(j, b, ahead=True, pref=True):
            gather_copy(b).wait()
            if ahead:
                b2 = (b + AH) % R
                idx_wait(j + AH, b2)
                gather_copy(b2).start()
            pltpu.sync_copy(rows_v.at[b], acc_sh.at[dring_v.at[b]], add=True)
            hist(b, ch, dring_v)
            if pref:
                idx_start(j + R, b)

        for j0 in range(R):
            idx_start(j0, j0)
        for j0 in range(AH):
            idx_wait(j0, j0)
            gather_copy(j0).start()

        ntrip = nfull // R
        npeel = nfull - R * (ntrip - 1)

        @pl.loop(0, ntrip - 1)
        def _(p):
            for b in range(R):
                body(R * p + b, b)

        for j in range(nfull - npeel, nfull):
            body(j, j % R, ahead=(j + AH < nfull), pref=(j + R < nfull))

        if tail_e:
            pltpu.sync_copy(src_hbm.at[pl.ds(base + nfull * ch, tail_e)],
                            sring_v.at[0].at[pl.ds(0, tail_e)])
            pltpu.sync_copy(dst_hbm.at[pl.ds(base + nfull * ch, tail_e)],
                            dtail_v.at[0])
            pltpu.async_copy(
                x_hbm.at[sring_v.at[0].at[pl.ds(0, tail_e)]],
                rows_v.at[0].at[pl.ds(0, tail_e)], gsems[0]).wait()
            pltpu.sync_copy(rows_v.at[0].at[pl.ds(0, tail_e)],
                            acc_sh.at[dtail_v.at[0]], add=True)
            hist(0, tail_e, dtail_v)

        plsc.subcore_barrier()
        pltpu.sync_copy(acc_sh.at[pl.ds(s * rps, rps)],
                        part_hbm.at[c].at[pl.ds(s * rps, rps)])

        @pl.when(s == 0)
        def _():
            pltpu.sync_copy(acc_sh.at[pl.ds(NS * rps, tail)],
                            part_hbm.at[c].at[pl.ds(NS * rps, tail)])

        pltpu.sync_copy(cnt_v, cnt_hbm.at[wid])

    return sc_agg(src, dst, x, zeros2d, zeros1d)


def _tc_combine(part_ref, cntp_ref, x_ref, wl_ref, bl_ref, wr_ref, g_ref,
                b_ref, o_ref):
    agg = part_ref[0] + part_ref[1]
    cnt = jnp.sum(cntp_ref[...], axis=1)
    mean = agg / jnp.maximum(cnt, 1.0)[:, None]
    h = (jnp.dot(mean, wl_ref[...], preferred_element_type=jnp.float32)
         + jnp.dot(x_ref[...], wr_ref[...], preferred_element_type=jnp.float32)
         + bl_ref[...])
    mu = jnp.mean(h, axis=1, keepdims=True)
    hc = h - mu
    var = jnp.mean(hc * hc, axis=1, keepdims=True)
    hn = hc * lax.rsqrt(var + 1e-5) * g_ref[...] + b_ref[...]
    o_ref[...] = 0.5 * hn * (1.0 + lax.erf(hn * (1.0 / math.sqrt(2.0))))


def kernel(x, edge_index, W_l, b_l, W_r, ln_gamma, ln_beta):
    n, d = x.shape
    src = edge_index[0]
    dst = edge_index[1]
    zeros2d = jnp.zeros((n, d), jnp.float32)
    zeros1d = jnp.zeros((n,), jnp.float32)
    part, cntp = _sc_aggregate(src, dst, x, zeros2d, zeros1d)

    blk = 1000
    grid = (n // blk,)
    out = pl.pallas_call(
        _tc_combine,
        grid=grid,
        in_specs=[
            pl.BlockSpec((NC, blk, d), lambda i: (0, i, 0)),
            pl.BlockSpec((blk, NW), lambda i: (i, 0)),
            pl.BlockSpec((blk, d), lambda i: (i, 0)),
            pl.BlockSpec((d, d), lambda i: (0, 0)),
            pl.BlockSpec((1, d), lambda i: (0, 0)),
            pl.BlockSpec((d, d), lambda i: (0, 0)),
            pl.BlockSpec((1, d), lambda i: (0, 0)),
            pl.BlockSpec((1, d), lambda i: (0, 0)),
        ],
        out_specs=pl.BlockSpec((blk, d), lambda i: (i, 0)),
        out_shape=jax.ShapeDtypeStruct((n, d), jnp.float32),
    )(part, cntp.T, x, W_l, b_l.reshape(1, d), W_r, ln_gamma.reshape(1, d),
      ln_beta.reshape(1, d))
    return out


# speedup vs baseline: 4.3964x; 1.2497x over previous
import dataclasses
import functools
import math

import jax
import jax.numpy as jnp
from jax import lax
from jax.experimental import pallas as pl
from jax.experimental.pallas import tpu as pltpu
from jax.experimental.pallas import tpu_sc as plsc

NC = 2
NS = 16
NW = NC * NS


def _sc_aggregate(src, dst, x, zeros2d, zeros1d):
    n, d = x.shape
    e = src.shape[0]
    epw = e // NW
    ch = 96
    nfull = epw // ch
    tail_e = epw - nfull * ch
    R = 3
    rps = (n // NS) // 8 * 8
    tail = n - NS * rps

    mesh = plsc.VectorSubcoreMesh(
        core_axis_name="c", subcore_axis_name="s", num_cores=NC,
        num_subcores=NS)

    cp = pltpu.CompilerParams()
    if "needs_layout_passes" in pltpu.CompilerParams.__dataclass_fields__:
        cp = dataclasses.replace(cp, needs_layout_passes=False)

    @functools.partial(
        pl.kernel,
        out_type=(
            jax.ShapeDtypeStruct((NC, n, d), jnp.float32),
            jax.ShapeDtypeStruct((NW, n), jnp.float32),
        ),
        mesh=mesh,
        scratch_types=[
            pltpu.VMEM((R, ch), jnp.int32),
            pltpu.VMEM((R, ch), jnp.int32),
            pltpu.VMEM((1, 16), jnp.int32),
            pltpu.VMEM((R, ch, d), jnp.float32),
            pltpu.VMEM((n,), jnp.float32),
            pltpu.VMEM_SHARED((n, d), jnp.float32),
        ] + [pltpu.SemaphoreType.DMA] * (2 * R),
        compiler_params=cp,
    )
    def sc_agg(src_hbm, dst_hbm, x_hbm, z2_hbm, z1_hbm, part_hbm, cnt_hbm,
               sring_v, dring_v, dtail_v, rows_v, cnt_v, acc_sh, *sems):
        c = lax.axis_index("c")
        s = lax.axis_index("s")
        wid = c * NS + s
        base = wid * epw
        pltpu.sync_copy(z1_hbm, cnt_v)
        pltpu.sync_copy(z2_hbm.at[pl.ds(s * rps, rps)],
                        acc_sh.at[pl.ds(s * rps, rps)])

        @pl.when(s == 0)
        def _():
            pltpu.sync_copy(z2_hbm.at[pl.ds(NS * rps, tail)],
                            acc_sh.at[pl.ds(NS * rps, tail)])

        plsc.subcore_barrier()

        ones = jnp.ones((16,), jnp.float32)
        gsems = sems[:R]
        isems = sems[R:2 * R]

        def hist(b, width, ring):
            for k in range(width // 16):
                dk = ring[b, pl.ds(k * 16, 16)]
                plsc.addupdate_scatter(cnt_v, [dk], ones)

        def idx_copy(j, b):
            return (
                pltpu.make_async_copy(src_hbm.at[pl.ds(base + j * ch, ch)],
                                      sring_v.at[b], isems[b]),
                pltpu.make_async_copy(dst_hbm.at[pl.ds(base + j * ch, ch)],
                                      dring_v.at[b], isems[b]),
            )

        def idx_start(j, b):
            for cp_ in idx_copy(j, b):
                cp_.start()

        def idx_wait(j, b):
            for cp_ in idx_copy(j, b):
                cp_.wait()

        def gather_copy(b):
            return pltpu.make_async_copy(
                x_hbm.at[sring_v.at[b]], rows_v.at[b], gsems[b])

        AH = R - 1

        def body(j, b, ahead=True, pref=True):
            gather_copy(b).wait()
            if ahead:
                b2 = (b + AH) % R
                idx_wait(j + AH, b2)
                gather_copy(b2).start()
            pltpu.sync_copy(rows_v.at[b], acc_sh.at[dring_v.at[b]], add=True)
            hist(b, ch, dring_v)
            if pref:
                idx_start(j + R, b)

        for j0 in range(R):
            idx_start(j0, j0)
        for j0 in range(AH):
            idx_wait(j0, j0)
            gather_copy(j0).start()

        ntrip = nfull // R
        npeel = nfull - R * (ntrip - 1)

        @pl.loop(0, ntrip - 1)
        def _(p):
            for b in range(R):
                body(R * p + b, b)

        for j in range(nfull - npeel, nfull):
            body(j, j % R, ahead=(j + AH < nfull), pref=(j + R < nfull))

        if tail_e:
            pltpu.sync_copy(src_hbm.at[pl.ds(base + nfull * ch, tail_e)],
                            sring_v.at[0].at[pl.ds(0, tail_e)])
            pltpu.sync_copy(dst_hbm.at[pl.ds(base + nfull * ch, tail_e)],
                            dtail_v.at[0])
            pltpu.async_copy(
                x_hbm.at[sring_v.at[0].at[pl.ds(0, tail_e)]],
                rows_v.at[0].at[pl.ds(0, tail_e)], gsems[0]).wait()
            pltpu.sync_copy(rows_v.at[0].at[pl.ds(0, tail_e)],
                            acc_sh.at[dtail_v.at[0]], add=True)
            hist(0, tail_e, dtail_v)

        plsc.subcore_barrier()
        pltpu.sync_copy(acc_sh.at[pl.ds(s * rps, rps)],
                        part_hbm.at[c].at[pl.ds(s * rps, rps)])

        @pl.when(s == 0)
        def _():
            pltpu.sync_copy(acc_sh.at[pl.ds(NS * rps, tail)],
                            part_hbm.at[c].at[pl.ds(NS * rps, tail)])

        pltpu.sync_copy(cnt_v, cnt_hbm.at[wid])

    return sc_agg(src, dst, x, zeros2d, zeros1d)


def _tc_combine(part_ref, cntp_ref, x_ref, wl_ref, bl_ref, wr_ref, g_ref,
                b_ref, o_ref):
    agg = part_ref[0] + part_ref[1]
    cnt = jnp.sum(cntp_ref[...], axis=1)
    mean = agg / jnp.maximum(cnt, 1.0)[:, None]
    h = (jnp.dot(mean, wl_ref[...], preferred_element_type=jnp.float32)
         + jnp.dot(x_ref[...], wr_ref[...], preferred_element_type=jnp.float32)
         + bl_ref[...])
    mu = jnp.mean(h, axis=1, keepdims=True)
    hc = h - mu
    var = jnp.mean(hc * hc, axis=1, keepdims=True)
    hn = hc * lax.rsqrt(var + 1e-5) * g_ref[...] + b_ref[...]
    o_ref[...] = 0.5 * hn * (1.0 + lax.erf(hn * (1.0 / math.sqrt(2.0))))


def kernel(x, edge_index, W_l, b_l, W_r, ln_gamma, ln_beta):
    n, d = x.shape
    src = edge_index[0]
    dst = edge_index[1]
    zeros2d = jnp.zeros((n, d), jnp.float32)
    zeros1d = jnp.zeros((n,), jnp.float32)
    part, cntp = _sc_aggregate(src, dst, x, zeros2d, zeros1d)

    blk = 1000
    grid = (n // blk,)
    out = pl.pallas_call(
        _tc_combine,
        grid=grid,
        in_specs=[
            pl.BlockSpec((NC, blk, d), lambda i: (0, i, 0)),
            pl.BlockSpec((blk, NW), lambda i: (i, 0)),
            pl.BlockSpec((blk, d), lambda i: (i, 0)),
            pl.BlockSpec((d, d), lambda i: (0, 0)),
            pl.BlockSpec((1, d), lambda i: (0, 0)),
            pl.BlockSpec((d, d), lambda i: (0, 0)),
            pl.BlockSpec((1, d), lambda i: (0, 0)),
            pl.BlockSpec((1, d), lambda i: (0, 0)),
        ],
        out_specs=pl.BlockSpec((blk, d), lambda i: (i, 0)),
        out_shape=jax.ShapeDtypeStruct((n, d), jnp.float32),
    )(part, cntp.T, x, W_l, b_l.reshape(1, d), W_r, ln_gamma.reshape(1, d),
      ln_beta.reshape(1, d))
    return out


# speedup vs baseline: 4.8501x; 1.1032x over previous
import dataclasses
import functools
import math

import jax
import jax.numpy as jnp
from jax import lax
from jax.experimental import pallas as pl
from jax.experimental.pallas import tpu as pltpu
from jax.experimental.pallas import tpu_sc as plsc

NC = 2
NS = 16
NW = NC * NS


def _sc_aggregate(ei, x):
    n, d = x.shape
    e = ei.shape[0] // 2
    epw = e // NW
    ch = 96
    nfull = epw // ch
    tail_e = epw - nfull * ch
    R = 3
    rps = (n // NS) // 8 * 8
    tail = n - NS * rps

    mesh = plsc.VectorSubcoreMesh(
        core_axis_name="c", subcore_axis_name="s", num_cores=NC,
        num_subcores=NS)

    cp = pltpu.CompilerParams()
    if "needs_layout_passes" in pltpu.CompilerParams.__dataclass_fields__:
        cp = dataclasses.replace(cp, needs_layout_passes=False)

    @functools.partial(
        pl.kernel,
        out_type=(
            jax.ShapeDtypeStruct((NC, n, d), jnp.float32),
            jax.ShapeDtypeStruct((NW, n), jnp.float32),
        ),
        mesh=mesh,
        scratch_types=[
            pltpu.VMEM((R, ch), jnp.int32),
            pltpu.VMEM((R, ch), jnp.int32),
            pltpu.VMEM((1, 16), jnp.int32),
            pltpu.VMEM((R, ch, d), jnp.float32),
            pltpu.VMEM((n,), jnp.float32),
            pltpu.VMEM_SHARED((n, d), jnp.float32),
        ] + [pltpu.SemaphoreType.DMA] * (2 * R),
        compiler_params=cp,
    )
    def sc_agg(ei_hbm, x_hbm, part_hbm, cnt_hbm,
               sring_v, dring_v, dtail_v, rows_v, cnt_v, acc_sh, *sems):
        c = lax.axis_index("c")
        s = lax.axis_index("s")
        wid = c * NS + s
        base = wid * epw

        zvec = jnp.zeros((16,), jnp.float32)

        @pl.loop(0, n // 16)
        def _(i):
            cnt_v[pl.ds(i * 16, 16)] = zvec

        @pl.loop(0, ch)
        def _(r):
            for k in range(d // 16):
                rows_v[0, r, pl.ds(k * 16, 16)] = zvec

        for m in range(rps // ch):
            pltpu.sync_copy(rows_v.at[0],
                            acc_sh.at[pl.ds(s * rps + m * ch, ch)])
        rem = rps - (rps // ch) * ch
        if rem:
            pltpu.sync_copy(rows_v.at[0].at[pl.ds(0, rem)],
                            acc_sh.at[pl.ds(s * rps + (rps // ch) * ch, rem)])

        @pl.when(s == 0)
        def _():
            pltpu.sync_copy(rows_v.at[0].at[pl.ds(0, tail)],
                            acc_sh.at[pl.ds(NS * rps, tail)])

        plsc.subcore_barrier()

        ones = jnp.ones((16,), jnp.float32)
        gsems = sems[:R]
        isems = sems[R:2 * R]

        def hist(b, width, ring):
            for k in range(width // 16):
                dk = ring[b, pl.ds(k * 16, 16)]
                plsc.addupdate_scatter(cnt_v, [dk], ones)

        def idx_copy(j, b):
            return (
                pltpu.make_async_copy(ei_hbm.at[pl.ds(base + j * ch, ch)],
                                      sring_v.at[b], isems[b]),
                pltpu.make_async_copy(ei_hbm.at[pl.ds(e + base + j * ch, ch)],
                                      dring_v.at[b], isems[b]),
            )

        def idx_start(j, b):
            for cp_ in idx_copy(j, b):
                cp_.start()

        def idx_wait(j, b):
            for cp_ in idx_copy(j, b):
                cp_.wait()

        def gather_copy(b):
            return pltpu.make_async_copy(
                x_hbm.at[sring_v.at[b]], rows_v.at[b], gsems[b])

        AH = R - 1

        def body(j, b, ahead=True, pref=True):
            gather_copy(b).wait()
            if ahead:
                b2 = (b + AH) % R
                idx_wait(j + AH, b2)
                gather_copy(b2).start()
            pltpu.sync_copy(rows_v.at[b], acc_sh.at[dring_v.at[b]], add=True)
            hist(b, ch, dring_v)
            if pref:
                idx_start(j + R, b)

        for j0 in range(R):
            idx_start(j0, j0)
        for j0 in range(AH):
            idx_wait(j0, j0)
            gather_copy(j0).start()

        ntrip = nfull // R
        npeel = nfull - R * (ntrip - 1)

        @pl.loop(0, ntrip - 1)
        def _(p):
            for b in range(R):
                body(R * p + b, b)

        for j in range(nfull - npeel, nfull):
            body(j, j % R, ahead=(j + AH < nfull), pref=(j + R < nfull))

        if tail_e:
            pltpu.sync_copy(ei_hbm.at[pl.ds(base + nfull * ch, tail_e)],
                            sring_v.at[0].at[pl.ds(0, tail_e)])
            pltpu.sync_copy(ei_hbm.at[pl.ds(e + base + nfull * ch, tail_e)],
                            dtail_v.at[0])
            pltpu.async_copy(
                x_hbm.at[sring_v.at[0].at[pl.ds(0, tail_e)]],
                rows_v.at[0].at[pl.ds(0, tail_e)], gsems[0]).wait()
            pltpu.sync_copy(rows_v.at[0].at[pl.ds(0, tail_e)],
                            acc_sh.at[dtail_v.at[0]], add=True)
            hist(0, tail_e, dtail_v)

        plsc.subcore_barrier()
        pltpu.sync_copy(acc_sh.at[pl.ds(s * rps, rps)],
                        part_hbm.at[c].at[pl.ds(s * rps, rps)])

        @pl.when(s == 0)
        def _():
            pltpu.sync_copy(acc_sh.at[pl.ds(NS * rps, tail)],
                            part_hbm.at[c].at[pl.ds(NS * rps, tail)])

        pltpu.sync_copy(cnt_v, cnt_hbm.at[wid])

    return sc_agg(ei, x)


def _tc_combine(part_ref, cntp_ref, x_ref, wl_ref, bl_ref, wr_ref, g_ref,
                b_ref, o_ref):
    agg = part_ref[0] + part_ref[1]
    cnt = jnp.sum(cntp_ref[...], axis=1)
    mean = agg / jnp.maximum(cnt, 1.0)[:, None]
    h = (jnp.dot(mean, wl_ref[...], preferred_element_type=jnp.float32)
         + jnp.dot(x_ref[...], wr_ref[...], preferred_element_type=jnp.float32)
         + bl_ref[...])
    mu = jnp.mean(h, axis=1, keepdims=True)
    hc = h - mu
    var = jnp.mean(hc * hc, axis=1, keepdims=True)
    hn = hc * lax.rsqrt(var + 1e-5) * g_ref[...] + b_ref[...]
    o_ref[...] = 0.5 * hn * (1.0 + lax.erf(hn * (1.0 / math.sqrt(2.0))))


def kernel(x, edge_index, W_l, b_l, W_r, ln_gamma, ln_beta):
    n, d = x.shape
    part, cntp = _sc_aggregate(edge_index.reshape(-1), x)

    blk = 1000
    grid = (n // blk,)
    out = pl.pallas_call(
        _tc_combine,
        grid=grid,
        in_specs=[
            pl.BlockSpec((NC, blk, d), lambda i: (0, i, 0)),
            pl.BlockSpec((blk, NW), lambda i: (i, 0)),
            pl.BlockSpec((blk, d), lambda i: (i, 0)),
            pl.BlockSpec((d, d), lambda i: (0, 0)),
            pl.BlockSpec((1, d), lambda i: (0, 0)),
            pl.BlockSpec((d, d), lambda i: (0, 0)),
            pl.BlockSpec((1, d), lambda i: (0, 0)),
            pl.BlockSpec((1, d), lambda i: (0, 0)),
        ],
        out_specs=pl.BlockSpec((blk, d), lambda i: (i, 0)),
        out_shape=jax.ShapeDtypeStruct((n, d), jnp.float32),
    )(part, cntp.T, x, W_l, b_l.reshape(1, d), W_r, ln_gamma.reshape(1, d),
      ln_beta.reshape(1, d))
    return out


# speedup vs baseline: 5.0390x; 1.0390x over previous
import dataclasses
import functools
import math

import jax
import jax.numpy as jnp
from jax import lax
from jax.experimental import pallas as pl
from jax.experimental.pallas import tpu as pltpu
from jax.experimental.pallas import tpu_sc as plsc

NC = 2
NS = 16
NW = NC * NS


def _sc_aggregate(ei, x):
    n, d = x.shape
    e = ei.shape[0] // 2
    epw = e // NW
    ch = 96
    nfull = epw // ch
    tail_e = epw - nfull * ch
    R = 3
    rps = (n // NS) // 8 * 8
    tail = n - NS * rps

    mesh = plsc.VectorSubcoreMesh(
        core_axis_name="c", subcore_axis_name="s", num_cores=NC,
        num_subcores=NS)

    cp = pltpu.CompilerParams()
    if "needs_layout_passes" in pltpu.CompilerParams.__dataclass_fields__:
        cp = dataclasses.replace(cp, needs_layout_passes=False)

    @functools.partial(
        pl.kernel,
        out_type=(
            jax.ShapeDtypeStruct((NC, n, d), jnp.float32),
            jax.ShapeDtypeStruct((NW, n), jnp.float32),
        ),
        mesh=mesh,
        scratch_types=[
            pltpu.VMEM((R, ch), jnp.int32),
            pltpu.VMEM((R, ch), jnp.int32),
            pltpu.VMEM((1, 16), jnp.int32),
            pltpu.VMEM((R, ch, d), jnp.float32),
            pltpu.VMEM((n,), jnp.float32),
            pltpu.VMEM_SHARED((n, d), jnp.float32),
        ] + [pltpu.SemaphoreType.DMA] * (2 * R),
        compiler_params=cp,
    )
    def sc_agg(ei_hbm, x_hbm, part_hbm, cnt_hbm,
               sring_v, dring_v, dtail_v, rows_v, cnt_v, acc_sh, *sems):
        c = lax.axis_index("c")
        s = lax.axis_index("s")
        wid = c * NS + s
        base = wid * epw

        zvec = jnp.zeros((16,), jnp.float32)

        @pl.loop(0, n // 16)
        def _(i):
            cnt_v[pl.ds(i * 16, 16)] = zvec

        @pl.loop(0, ch)
        def _(r):
            for k in range(d // 16):
                rows_v[0, r, pl.ds(k * 16, 16)] = zvec

        for m in range(rps // ch):
            pltpu.sync_copy(rows_v.at[0],
                            acc_sh.at[pl.ds(s * rps + m * ch, ch)])
        rem = rps - (rps // ch) * ch
        if rem:
            pltpu.sync_copy(rows_v.at[0].at[pl.ds(0, rem)],
                            acc_sh.at[pl.ds(s * rps + (rps // ch) * ch, rem)])

        @pl.when(s == 0)
        def _():
            pltpu.sync_copy(rows_v.at[0].at[pl.ds(0, tail)],
                            acc_sh.at[pl.ds(NS * rps, tail)])

        plsc.subcore_barrier()

        ones = jnp.ones((16,), jnp.float32)
        gsems = sems[:R]
        isems = sems[R:2 * R]

        def hist(b, width, ring):
            for k in range(width // 16):
                dk = ring[b, pl.ds(k * 16, 16)]
                plsc.addupdate_scatter(cnt_v, [dk], ones)

        def idx_copy(j, b):
            return (
                pltpu.make_async_copy(ei_hbm.at[pl.ds(base + j * ch, ch)],
                                      sring_v.at[b], isems[b]),
                pltpu.make_async_copy(ei_hbm.at[pl.ds(e + base + j * ch, ch)],
                                      dring_v.at[b], isems[b]),
            )

        def idx_start(j, b):
            for cp_ in idx_copy(j, b):
                cp_.start()

        def idx_wait(j, b):
            for cp_ in idx_copy(j, b):
                cp_.wait()

        def gather_copy(b):
            return pltpu.make_async_copy(
                x_hbm.at[sring_v.at[b]], rows_v.at[b], gsems[b])

        AH = R - 1

        def body(j, b, ahead=True, pref=True):
            gather_copy(b).wait()
            if ahead:
                b2 = (b + AH) % R
                idx_wait(j + AH, b2)
                gather_copy(b2).start()
            pltpu.sync_copy(rows_v.at[b], acc_sh.at[dring_v.at[b]], add=True)
            hist(b, ch, dring_v)
            if pref:
                idx_start(j + R, b)

        for j0 in range(R):
            idx_start(j0, j0)
        for j0 in range(AH):
            idx_wait(j0, j0)
            gather_copy(j0).start()

        ntrip = nfull // R
        npeel = nfull - R * (ntrip - 1)

        @pl.loop(0, ntrip - 1)
        def _(p):
            for b in range(R):
                body(R * p + b, b)

        for j in range(nfull - npeel, nfull):
            body(j, j % R, ahead=(j + AH < nfull), pref=(j + R < nfull))

        if tail_e:
            pltpu.sync_copy(ei_hbm.at[pl.ds(base + nfull * ch, tail_e)],
                            sring_v.at[0].at[pl.ds(0, tail_e)])
            pltpu.sync_copy(ei_hbm.at[pl.ds(e + base + nfull * ch, tail_e)],
                            dtail_v.at[0])
            pltpu.async_copy(
                x_hbm.at[sring_v.at[0].at[pl.ds(0, tail_e)]],
                rows_v.at[0].at[pl.ds(0, tail_e)], gsems[0]).wait()
            pltpu.sync_copy(rows_v.at[0].at[pl.ds(0, tail_e)],
                            acc_sh.at[dtail_v.at[0]], add=True)
            hist(0, tail_e, dtail_v)

        plsc.subcore_barrier()
        pltpu.sync_copy(acc_sh.at[pl.ds(s * rps, rps)],
                        part_hbm.at[c].at[pl.ds(s * rps, rps)])

        @pl.when(s == 0)
        def _():
            pltpu.sync_copy(acc_sh.at[pl.ds(NS * rps, tail)],
                            part_hbm.at[c].at[pl.ds(NS * rps, tail)])

        pltpu.sync_copy(cnt_v, cnt_hbm.at[wid])

    return sc_agg(ei, x)


def _tc_combine(part_ref, cntp_ref, x_ref, wl_ref, bl_ref, wr_ref, g_ref,
                b_ref, o_ref):
    agg = part_ref[0] + part_ref[1]
    cnt = jnp.sum(cntp_ref[...], axis=0)
    mean = agg / jnp.maximum(cnt, 1.0)[:, None]
    h = (jnp.dot(mean, wl_ref[...], preferred_element_type=jnp.float32)
         + jnp.dot(x_ref[...], wr_ref[...], preferred_element_type=jnp.float32)
         + bl_ref[...])
    mu = jnp.mean(h, axis=1, keepdims=True)
    hc = h - mu
    var = jnp.mean(hc * hc, axis=1, keepdims=True)
    hn = hc * lax.rsqrt(var + 1e-5) * g_ref[...] + b_ref[...]
    o_ref[...] = 0.5 * hn * (1.0 + lax.erf(hn * (1.0 / math.sqrt(2.0))))


def kernel(x, edge_index, W_l, b_l, W_r, ln_gamma, ln_beta):
    n, d = x.shape
    part, cntp = _sc_aggregate(edge_index.reshape(-1), x)

    out = pl.pallas_call(
        _tc_combine,
        out_shape=jax.ShapeDtypeStruct((n, d), jnp.float32),
    )(part, cntp, x, W_l, b_l.reshape(1, d), W_r, ln_gamma.reshape(1, d),
      ln_beta.reshape(1, d))
    return out


# speedup vs baseline: 5.0460x; 1.0014x over previous
import dataclasses
import functools
import math

import jax
import jax.numpy as jnp
from jax import lax
from jax.experimental import pallas as pl
from jax.experimental.pallas import tpu as pltpu
from jax.experimental.pallas import tpu_sc as plsc

NC = 2
NS = 16
NW = NC * NS


def _sc_aggregate(ei, x):
    n, d = x.shape
    e = ei.shape[0] // 2
    epw = e // NW
    ch = 128
    nfull = epw // ch
    tail_e = epw - nfull * ch
    R = 2
    rps = (n // NS) // 8 * 8
    tail = n - NS * rps

    mesh = plsc.VectorSubcoreMesh(
        core_axis_name="c", subcore_axis_name="s", num_cores=NC,
        num_subcores=NS)

    cp = pltpu.CompilerParams()
    if "needs_layout_passes" in pltpu.CompilerParams.__dataclass_fields__:
        cp = dataclasses.replace(cp, needs_layout_passes=False)

    @functools.partial(
        pl.kernel,
        out_type=(
            jax.ShapeDtypeStruct((NC, n, d), jnp.float32),
            jax.ShapeDtypeStruct((NW, n), jnp.float32),
        ),
        mesh=mesh,
        scratch_types=[
            pltpu.VMEM((R, ch), jnp.int32),
            pltpu.VMEM((R, ch), jnp.int32),
            pltpu.VMEM((1, 16), jnp.int32),
            pltpu.VMEM((R, ch, d), jnp.float32),
            pltpu.VMEM((n,), jnp.float32),
            pltpu.VMEM_SHARED((n, d), jnp.float32),
        ] + [pltpu.SemaphoreType.DMA] * (2 * R),
        compiler_params=cp,
    )
    def sc_agg(ei_hbm, x_hbm, part_hbm, cnt_hbm,
               sring_v, dring_v, dtail_v, rows_v, cnt_v, acc_sh, *sems):
        c = lax.axis_index("c")
        s = lax.axis_index("s")
        wid = c * NS + s
        base = wid * epw

        zvec = jnp.zeros((16,), jnp.float32)

        @pl.loop(0, n // 16)
        def _(i):
            cnt_v[pl.ds(i * 16, 16)] = zvec

        @pl.loop(0, ch)
        def _(r):
            for k in range(d // 16):
                rows_v[0, r, pl.ds(k * 16, 16)] = zvec

        for m in range(rps // ch):
            pltpu.sync_copy(rows_v.at[0],
                            acc_sh.at[pl.ds(s * rps + m * ch, ch)])
        rem = rps - (rps // ch) * ch
        if rem:
            pltpu.sync_copy(rows_v.at[0].at[pl.ds(0, rem)],
                            acc_sh.at[pl.ds(s * rps + (rps // ch) * ch, rem)])

        @pl.when(s == 0)
        def _():
            pltpu.sync_copy(rows_v.at[0].at[pl.ds(0, tail)],
                            acc_sh.at[pl.ds(NS * rps, tail)])

        plsc.subcore_barrier()

        ones = jnp.ones((16,), jnp.float32)
        gsems = sems[:R]
        isems = sems[R:2 * R]

        def hist(b, width, ring):
            for k in range(width // 16):
                dk = ring[b, pl.ds(k * 16, 16)]
                plsc.addupdate_scatter(cnt_v, [dk], ones)

        def idx_copy(j, b):
            return (
                pltpu.make_async_copy(ei_hbm.at[pl.ds(base + j * ch, ch)],
                                      sring_v.at[b], isems[b]),
                pltpu.make_async_copy(ei_hbm.at[pl.ds(e + base + j * ch, ch)],
                                      dring_v.at[b], isems[b]),
            )

        def idx_start(j, b):
            for cp_ in idx_copy(j, b):
                cp_.start()

        def idx_wait(j, b):
            for cp_ in idx_copy(j, b):
                cp_.wait()

        def gather_copy(b):
            return pltpu.make_async_copy(
                x_hbm.at[sring_v.at[b]], rows_v.at[b], gsems[b])

        AH = R - 1

        def body(j, b, ahead=True, pref=True):
            gather_copy(b).wait()
            if ahead:
                b2 = (b + AH) % R
                idx_wait(j + AH, b2)
                gather_copy(b2).start()
            pltpu.sync_copy(rows_v.at[b], acc_sh.at[dring_v.at[b]], add=True)
            hist(b, ch, dring_v)
            if pref:
                idx_start(j + R, b)

        for j0 in range(R):
            idx_start(j0, j0)
        for j0 in range(AH):
            idx_wait(j0, j0)
            gather_copy(j0).start()

        ntrip = nfull // R
        npeel = nfull - R * (ntrip - 1)

        @pl.loop(0, ntrip - 1)
        def _(p):
            for b in range(R):
                body(R * p + b, b)

        for j in range(nfull - npeel, nfull):
            body(j, j % R, ahead=(j + AH < nfull), pref=(j + R < nfull))

        if tail_e:
            pltpu.sync_copy(ei_hbm.at[pl.ds(base + nfull * ch, tail_e)],
                            sring_v.at[0].at[pl.ds(0, tail_e)])
            pltpu.sync_copy(ei_hbm.at[pl.ds(e + base + nfull * ch, tail_e)],
                            dtail_v.at[0])
            pltpu.async_copy(
                x_hbm.at[sring_v.at[0].at[pl.ds(0, tail_e)]],
                rows_v.at[0].at[pl.ds(0, tail_e)], gsems[0]).wait()
            pltpu.sync_copy(rows_v.at[0].at[pl.ds(0, tail_e)],
                            acc_sh.at[dtail_v.at[0]], add=True)
            hist(0, tail_e, dtail_v)

        plsc.subcore_barrier()
        pltpu.sync_copy(acc_sh.at[pl.ds(s * rps, rps)],
                        part_hbm.at[c].at[pl.ds(s * rps, rps)])

        @pl.when(s == 0)
        def _():
            pltpu.sync_copy(acc_sh.at[pl.ds(NS * rps, tail)],
                            part_hbm.at[c].at[pl.ds(NS * rps, tail)])

        pltpu.sync_copy(cnt_v, cnt_hbm.at[wid])

    return sc_agg(ei, x)


def _tc_combine(part_ref, cntp_ref, x_ref, wl_ref, bl_ref, wr_ref, g_ref,
                b_ref, o_ref):
    agg = part_ref[0] + part_ref[1]
    cnt = jnp.sum(cntp_ref[...], axis=0)
    mean = agg / jnp.maximum(cnt, 1.0)[:, None]
    h = (jnp.dot(mean, wl_ref[...], preferred_element_type=jnp.float32)
         + jnp.dot(x_ref[...], wr_ref[...], preferred_element_type=jnp.float32)
         + bl_ref[...])
    mu = jnp.mean(h, axis=1, keepdims=True)
    hc = h - mu
    var = jnp.mean(hc * hc, axis=1, keepdims=True)
    hn = hc * lax.rsqrt(var + 1e-5) * g_ref[...] + b_ref[...]
    o_ref[...] = 0.5 * hn * (1.0 + lax.erf(hn * (1.0 / math.sqrt(2.0))))


def kernel(x, edge_index, W_l, b_l, W_r, ln_gamma, ln_beta):
    n, d = x.shape
    part, cntp = _sc_aggregate(edge_index.reshape(-1), x)

    out = pl.pallas_call(
        _tc_combine,
        out_shape=jax.ShapeDtypeStruct((n, d), jnp.float32),
    )(part, cntp, x, W_l, b_l.reshape(1, d), W_r, ln_gamma.reshape(1, d),
      ln_beta.reshape(1, d))
    return out
